# baseline (speedup 1.0000x reference)
import functools

import jax
import jax.numpy as jnp
from jax import lax
from jax.experimental import pallas as pl
from jax.experimental.pallas import tpu as pltpu
from jax.experimental.pallas import tpu_sc as plsc

VOCAB = 1000000
EMBED = 64
B = 16384
L = 50
NUM_CLASSES = 20

NC = 2
NS = 16
NW = NC * NS
BAGS_PER_W = B // NW
IDX_PER_W = BAGS_PER_W * L
GROUP_BAGS = 4
GROUP_IDX = GROUP_BAGS * L
NGROUPS = BAGS_PER_W // GROUP_BAGS
CHUNK0 = 104
CHUNK1 = GROUP_IDX - CHUNK0
VREGS = EMBED // 16


def _sc_body(data_ref, table_ref, out_ref, idx_v, rows0, rows1, pooled_v,
             sem0, sem1):
    wid = lax.axis_index("s") * NC + lax.axis_index("c")
    pltpu.sync_copy(data_ref.at[pl.ds(wid * IDX_PER_W, IDX_PER_W)], idx_v)

    bufs = (rows0, rows1)
    sems = (sem0, sem1)

    def gather_descrs(g, slot):
        off = pl.multiple_of(g * GROUP_IDX, 8)
        buf, sem = bufs[slot], sems[slot]
        return (
            pltpu.make_async_copy(
                table_ref.at[idx_v.at[pl.ds(off, CHUNK0)]],
                buf.at[pl.ds(0, CHUNK0)], sem),
            pltpu.make_async_copy(
                table_ref.at[idx_v.at[pl.ds(off + CHUNK0, CHUNK1)]],
                buf.at[pl.ds(CHUNK0, CHUNK1)], sem),
        )

    def issue(g, slot):
        for d in gather_descrs(g, slot):
            d.start()

    def drain(g, slot):
        for d in gather_descrs(g, slot):
            d.wait()

    def compute(g, slot):
        buf = bufs[slot]
        for bag in range(GROUP_BAGS):
            def rbody(r, acc):
                row = bag * L + r
                return tuple(acc[k] + buf[row, pl.ds(k * 16, 16)]
                             for k in range(VREGS))
            acc = lax.fori_loop(
                0, L, rbody,
                tuple(jnp.zeros((16,), jnp.float32) for _ in range(VREGS)),
                unroll=10)
            for k in range(VREGS):
                pooled_v[g * GROUP_BAGS + bag, pl.ds(k * 16, 16)] = acc[k]

    issue(0, 0)

    def outer(g2, carry):
        for b in range(2):
            g = g2 * 2 + b

            @pl.when(g + 1 < NGROUPS)
            def _():
                issue(g + 1, 1 - b)

            drain(g, b)
            compute(g, b)
        return carry

    lax.fori_loop(0, NGROUPS // 2, outer, 0)

    pltpu.sync_copy(pooled_v, out_ref.at[pl.ds(wid * BAGS_PER_W, BAGS_PER_W)])


def _tc_head(pooled_ref, w_ref, b_ref, out_ref):
    out_ref[...] = (
        jnp.dot(pooled_ref[...], w_ref[...],
                preferred_element_type=jnp.float32)
        + b_ref[...]
    )


@jax.jit
def kernel(data, table, W, b):
    data_flat = data.reshape(-1)

    sc_pool = pl.kernel(
        _sc_body,
        out_type=jax.ShapeDtypeStruct((B, EMBED), jnp.float32),
        mesh=plsc.VectorSubcoreMesh(
            core_axis_name="c", subcore_axis_name="s",
            num_cores=NC, num_subcores=NS),
        scratch_types=[
            pltpu.VMEM((IDX_PER_W,), jnp.int32),
            pltpu.VMEM((GROUP_IDX, EMBED), jnp.float32),
            pltpu.VMEM((GROUP_IDX, EMBED), jnp.float32),
            pltpu.VMEM((BAGS_PER_W, EMBED), jnp.float32),
            pltpu.SemaphoreType.DMA,
            pltpu.SemaphoreType.DMA,
        ],
        compiler_params=pltpu.CompilerParams(use_tc_tiling_on_sc=False),
    )
    pooled_sum = sc_pool(data_flat, table)

    w_pad = jnp.zeros((EMBED, 128), jnp.float32)
    w_pad = lax.dynamic_update_slice(w_pad, W.T * (1.0 / L), (0, 0))
    b_pad = jnp.zeros((1, 128), jnp.float32)
    b_pad = lax.dynamic_update_slice(b_pad, b[None, :], (0, 0))

    blk = 2048
    logits_pad = pl.pallas_call(
        _tc_head,
        grid=(B // blk,),
        in_specs=[
            pl.BlockSpec((blk, EMBED), lambda i: (i, 0)),
            pl.BlockSpec((EMBED, 128), lambda i: (0, 0)),
            pl.BlockSpec((1, 128), lambda i: (0, 0)),
        ],
        out_specs=pl.BlockSpec((blk, 128), lambda i: (i, 0)),
        out_shape=jax.ShapeDtypeStruct((B, 128), jnp.float32),
    )(pooled_sum, w_pad, b_pad)

    return logits_pad[:, :NUM_CLASSES]


# speedup vs baseline: 1.0301x; 1.0301x over previous
import functools

import jax
import jax.numpy as jnp
from jax import lax
from jax.experimental import pallas as pl
from jax.experimental.pallas import tpu as pltpu
from jax.experimental.pallas import tpu_sc as plsc

VOCAB = 1000000
EMBED = 64
B = 16384
L = 50
NUM_CLASSES = 20

NC = 2
NS = 16
NW = NC * NS
BAGS_PER_W = B // NW
IDX_PER_W = BAGS_PER_W * L
GROUP_BAGS = 8
GROUP_IDX = GROUP_BAGS * L
NGROUPS = BAGS_PER_W // GROUP_BAGS
CHUNKS = ((0, 104), (104, 96), (200, 104), (304, 96))
VREGS = EMBED // 16


def _sc_body(data_ref, table_ref, out_ref, idx_v, rows0, rows1, pooled_v,
             sem0, sem1):
    wid = lax.axis_index("s") * NC + lax.axis_index("c")
    pltpu.sync_copy(data_ref.at[pl.ds(wid * IDX_PER_W, IDX_PER_W)], idx_v)

    bufs = (rows0, rows1)
    sems = (sem0, sem1)

    def gather_descrs(g, slot):
        off = pl.multiple_of(g * GROUP_IDX, 8)
        buf, sem = bufs[slot], sems[slot]
        return tuple(
            pltpu.make_async_copy(
                table_ref.at[idx_v.at[pl.ds(off + co, cn)]],
                buf.at[pl.ds(co, cn)], sem)
            for co, cn in CHUNKS)

    def issue(g, slot):
        for d in gather_descrs(g, slot):
            d.start()

    def drain(g, slot):
        for d in gather_descrs(g, slot):
            d.wait()

    def compute(g, slot):
        buf = bufs[slot]
        for bag in range(GROUP_BAGS):
            def rbody(r, acc):
                row = bag * L + r
                return tuple(acc[k] + buf[row, pl.ds(k * 16, 16)]
                             for k in range(VREGS))
            acc = lax.fori_loop(
                0, L, rbody,
                tuple(jnp.zeros((16,), jnp.float32) for _ in range(VREGS)),
                unroll=5)
            for k in range(VREGS):
                pooled_v[g * GROUP_BAGS + bag, pl.ds(k * 16, 16)] = acc[k]

    issue(0, 0)

    def outer(g2, carry):
        for b in range(2):
            g = g2 * 2 + b

            @pl.when(g + 1 < NGROUPS)
            def _():
                issue(g + 1, 1 - b)

            drain(g, b)
            compute(g, b)
        return carry

    lax.fori_loop(0, NGROUPS // 2, outer, 0)

    pltpu.sync_copy(pooled_v, out_ref.at[pl.ds(wid * BAGS_PER_W, BAGS_PER_W)])


def _tc_head(pooled_ref, w_ref, b_ref, out_ref):
    out_ref[...] = (
        jnp.dot(pooled_ref[...], w_ref[...],
                preferred_element_type=jnp.float32)
        + b_ref[...]
    )


@jax.jit
def kernel(data, table, W, b):
    data_flat = data.reshape(-1)

    sc_pool = pl.kernel(
        _sc_body,
        out_type=jax.ShapeDtypeStruct((B, EMBED), jnp.float32),
        mesh=plsc.VectorSubcoreMesh(
            core_axis_name="c", subcore_axis_name="s",
            num_cores=NC, num_subcores=NS),
        scratch_types=[
            pltpu.VMEM((IDX_PER_W,), jnp.int32),
            pltpu.VMEM((GROUP_IDX, EMBED), jnp.float32),
            pltpu.VMEM((GROUP_IDX, EMBED), jnp.float32),
            pltpu.VMEM((BAGS_PER_W, EMBED), jnp.float32),
            pltpu.SemaphoreType.DMA,
            pltpu.SemaphoreType.DMA,
        ],
        compiler_params=pltpu.CompilerParams(use_tc_tiling_on_sc=False),
    )
    pooled_sum = sc_pool(data_flat, table)

    w_pad = jnp.zeros((EMBED, 128), jnp.float32)
    w_pad = lax.dynamic_update_slice(w_pad, W.T * (1.0 / L), (0, 0))
    b_pad = jnp.zeros((1, 128), jnp.float32)
    b_pad = lax.dynamic_update_slice(b_pad, b[None, :], (0, 0))

    blk = 2048
    logits_pad = pl.pallas_call(
        _tc_head,
        grid=(B // blk,),
        in_specs=[
            pl.BlockSpec((blk, EMBED), lambda i: (i, 0)),
            pl.BlockSpec((EMBED, 128), lambda i: (0, 0)),
            pl.BlockSpec((1, 128), lambda i: (0, 0)),
        ],
        out_specs=pl.BlockSpec((blk, 128), lambda i: (i, 0)),
        out_shape=jax.ShapeDtypeStruct((B, 128), jnp.float32),
    )(pooled_sum, w_pad, b_pad)

    return logits_pad[:, :NUM_CLASSES]
